# Initial kernel scaffold; baseline (speedup 1.0000x reference)
import jax
import jax.numpy as jnp
from jax.experimental import pallas as pl


def kernel(queries, keys, values, Wq, Wk, Wv, w_out, tau, delta):
    raise NotImplementedError("write your pallas kernel here")


# speedup vs baseline: 5.6855x; 5.6855x over previous
import numpy as np
import jax
import jax.numpy as jnp
from jax.experimental import pallas as pl
from jax.experimental.pallas import tpu as pltpu

_B, _T, _L, _D, _H, _E = 2, 4, 2048, 768, 12, 64
_U = 40
_IDX_SAMPLE = np.random.default_rng(0).choice(_L, _U, replace=False)
_CH = 512
_NC = _L // _CH
_NEG = float("-inf")


def _proj_kernel(q_ref, v_ref, vs_ref, wq_ref, wv_ref, wsel_ref,
                 vout_ref, q0_ref, idx_ref, ob_ref, m_scr, carry_scr):
    c = pl.program_id(1)
    prec = jax.lax.Precision.HIGHEST
    prec_m = None

    q = q_ref[0, 0]
    Qc = jnp.dot(q, wq_ref[...], preferred_element_type=jnp.float32,
                 precision=prec_m)
    v = v_ref[0, 0]
    Vc = jnp.dot(v, wv_ref[...], preferred_element_type=jnp.float32,
                 precision=prec_m)
    vout_ref[0, 0] = Vc
    q0_ref[0, 0] = Qc

    Ks = jnp.dot(vs_ref[0, 0], wv_ref[...], preferred_element_type=jnp.float32,
                 precision=prec_m)

    @pl.when(c == 0)
    def _():
        carry_scr[...] = jnp.zeros_like(carry_scr)

    for h in range(_H):
        Qh = Qc[:, h * _E:(h + 1) * _E]
        Ksh = Ks[:, h * _E:(h + 1) * _E]
        S = jax.lax.dot_general(Qh, Ksh, (((1,), (1,)), ((), ())),
                                preferred_element_type=jnp.float32,
                                precision=prec_m)
        Mrow = jnp.max(S, axis=1) - jnp.sum(S, axis=1) * (1.0 / _L)
        m_scr[h, pl.ds(c * _CH, _CH)] = Mrow

    tri = (jax.lax.broadcasted_iota(jnp.int32, (_CH, _CH), 0)
           >= jax.lax.broadcasted_iota(jnp.int32, (_CH, _CH), 1)
           ).astype(jnp.float32)
    cum = jnp.dot(tri, Vc, preferred_element_type=jnp.float32,
                  precision=prec) + carry_scr[0:1, :]
    carry_scr[0:1, :] = carry_scr[0:1, :] + jnp.sum(Vc, axis=0, keepdims=True)
    ob_ref[0, 0] = jnp.dot(cum, wsel_ref[...],
                           preferred_element_type=jnp.float32, precision=prec)

    @pl.when(c == _NC - 1)
    def _():
        rows = jax.lax.broadcasted_iota(jnp.int32, (16, _L), 0)
        lanes = jax.lax.broadcasted_iota(jnp.int32, (16, _L), 1)
        cols = jax.lax.broadcasted_iota(jnp.int32, (16, 128), 1)
        M = jnp.where(rows < _H, m_scr[...], _NEG)

        def body(j, state):
            M, acc = state
            mval = jnp.max(M, axis=1, keepdims=True)
            cand = jnp.where(M == mval, lanes, _L)
            idx = jnp.min(cand, axis=1, keepdims=True)
            acc = jnp.where(cols == j, idx, acc)
            M = jnp.where(lanes == idx, _NEG, M)
            return M, acc

        _, acc = jax.lax.fori_loop(0, _U, body,
                                   (M, jnp.zeros((16, 128), jnp.int32)))
        idx_ref[0, 0] = acc


def _fix_kernel(v0_ref, v1_ref, q0_ref, i0_ref, i1_ref, ob_ref, w_ref,
                s_ref, out_ref):
    g = pl.program_id(1)
    prec = jax.lax.Precision.HIGHEST
    V0 = v0_ref[0, 0]
    V1 = v1_ref[0, 0]
    Q0 = q0_ref[0, 0]
    tau = s_ref[0]
    delta = s_ref[1]
    scale = jnp.float32(1.0 / np.sqrt(_E))

    tri = (jax.lax.broadcasted_iota(jnp.int32, (_CH, _CH), 0)
           >= jax.lax.broadcasted_iota(jnp.int32, (_CH, _CH), 1)
           ).astype(jnp.float32)
    carry = jnp.zeros((1, 128), jnp.float32)
    chunks = []
    for cc in range(_NC):
        vc = V0[cc * _CH:(cc + 1) * _CH, :]
        chunks.append(jnp.dot(tri, vc, preferred_element_type=jnp.float32,
                              precision=prec) + carry)
        carry = carry + jnp.sum(vc, axis=0, keepdims=True)
    cum = jnp.concatenate(chunks, axis=0)

    acc = jnp.zeros((_L, _E), jnp.float32)
    for hh in range(2):
        h = 2 * g + hh
        wh = w_ref[h]
        sl = slice(hh * _E, (hh + 1) * _E)
        Q0h = Q0[:, sl]
        cumh = cum[:, sl]
        Ps, attns = [], []
        for V, iref in ((V0, i0_ref), (V1, i1_ref)):
            idx_row = iref[0, 0, pl.ds(h, 1), :]
            idx_col = idx_row[0, :_U].reshape(_U, 1)
            P = (jax.lax.broadcasted_iota(jnp.int32, (_U, _L), 1)
                 == idx_col).astype(jnp.float32)
            Vh = V[:, sl]
            Qr = jnp.dot(P, Q0h, preferred_element_type=jnp.float32,
                         precision=prec)
            S = jax.lax.dot_general(Qr, Vh, (((1,), (1,)), ((), ())),
                                    preferred_element_type=jnp.float32,
                                    precision=prec)
            S = (S * tau + delta) * scale
            S = S - jnp.max(S, axis=1, keepdims=True)
            Sexp = jnp.exp(S)
            A = Sexp / jnp.sum(Sexp, axis=1, keepdims=True)
            attns.append(jnp.dot(A, Vh, preferred_element_type=jnp.float32,
                                 precision=prec))
            Ps.append(P)
        m0 = jnp.sum(Ps[0], axis=0)
        m1 = jnp.sum(Ps[1], axis=0)
        u = m0 + m1 - m0 * m1
        t0 = jax.lax.dot_general(Ps[0], attns[0], (((0,), (0,)), ((), ())),
                                 preferred_element_type=jnp.float32,
                                 precision=prec)
        t1 = jax.lax.dot_general(Ps[1], attns[1], (((0,), (0,)), ((), ())),
                                 preferred_element_type=jnp.float32,
                                 precision=prec)
        term = t0 * (1.0 - m1)[:, None] + t1 - u[:, None] * cumh
        acc = acc + wh * term
    @pl.when(g == 0)
    def _():
        out_ref[0] = ob_ref[0, 0]
    out_ref[0] += acc


def kernel(queries, keys, values, Wq, Wk, Wv, w_out, tau, delta):
    del keys, Wk
    wq2 = Wq.reshape(_D, _H * _E)
    wv2 = Wv.reshape(_D, _H * _E)
    vs = values[:, :, _IDX_SAMPLE, :]
    wsel = (w_out[:, None, None]
            * jnp.eye(_E, dtype=jnp.float32)[None]).reshape(_H * _E, _E)
    w_pad = jnp.concatenate([w_out, jnp.zeros((4,), jnp.float32)])
    scl = jnp.concatenate([tau, delta]).astype(jnp.float32)

    def bmap(p):
        return 1 - p // _T

    def tmap(p):
        return p % _T

    V, Q0, topidx, out_base = pl.pallas_call(
        _proj_kernel,
        grid=(_B * _T, _NC),
        in_specs=[
            pl.BlockSpec((1, 1, _CH, _D), lambda p, c: (bmap(p), tmap(p), c, 0)),
            pl.BlockSpec((1, 1, _CH, _D), lambda p, c: (bmap(p), tmap(p), c, 0)),
            pl.BlockSpec((1, 1, _U, _D), lambda p, c: (bmap(p), tmap(p), 0, 0)),
            pl.BlockSpec((_D, _H * _E), lambda p, c: (0, 0)),
            pl.BlockSpec((_D, _H * _E), lambda p, c: (0, 0)),
            pl.BlockSpec((_H * _E, _E), lambda p, c: (0, 0)),
        ],
        out_specs=[
            pl.BlockSpec((1, 1, _CH, _H * _E),
                         lambda p, c: (bmap(p), tmap(p), c, 0)),
            pl.BlockSpec((1, 1, _CH, _H * _E),
                         lambda p, c: (bmap(p), tmap(p), c, 0)),
            pl.BlockSpec((1, 1, 16, 128), lambda p, c: (bmap(p), tmap(p), 0, 0)),
            pl.BlockSpec((1, 1, _CH, _E), lambda p, c: (bmap(p), tmap(p), c, 0)),
        ],
        out_shape=[
            jax.ShapeDtypeStruct((_B, _T, _L, _H * _E), jnp.float32),
            jax.ShapeDtypeStruct((_B, _T, _L, _H * _E), jnp.float32),
            jax.ShapeDtypeStruct((_B, _T, 16, 128), jnp.int32),
            jax.ShapeDtypeStruct((_B, _T, _L, _E), jnp.float32),
        ],
        scratch_shapes=[
            pltpu.VMEM((16, _L), jnp.float32),
            pltpu.VMEM((8, _H * _E), jnp.float32),
        ],
    )(queries, values, vs, wq2, wv2, wsel)

    out0 = pl.pallas_call(
        _fix_kernel,
        grid=(_T, _H // 2),
        in_specs=[
            pl.BlockSpec((1, 1, _L, 128), lambda t, g: (0, t, 0, g)),
            pl.BlockSpec((1, 1, _L, 128), lambda t, g: (1, t, 0, g)),
            pl.BlockSpec((1, 1, _L, 128), lambda t, g: (0, t, 0, g)),
            pl.BlockSpec((1, 1, 16, 128), lambda t, g: (0, t, 0, 0)),
            pl.BlockSpec((1, 1, 16, 128), lambda t, g: (1, t, 0, 0)),
            pl.BlockSpec((1, 1, _L, _E), lambda t, g: (0, t, 0, 0)),
            pl.BlockSpec(memory_space=pltpu.SMEM),
            pl.BlockSpec(memory_space=pltpu.SMEM),
        ],
        out_specs=pl.BlockSpec((1, _L, _E), lambda t, g: (t, 0, 0)),
        out_shape=jax.ShapeDtypeStruct((_T, _L, _E), jnp.float32),
    )(V, V, Q0, topidx, topidx, out_base, w_pad, scl)

    return jnp.concatenate([out0[None], out_base[1:]], axis=0)


# speedup vs baseline: 9.8973x; 1.7408x over previous
import numpy as np
import jax
import jax.numpy as jnp
from jax.experimental import pallas as pl
from jax.experimental.pallas import tpu as pltpu

_B, _T, _L, _D, _H, _E = 2, 4, 2048, 768, 12, 64
_U = 40
_IDX_SAMPLE = np.random.default_rng(0).choice(_L, _U, replace=False)
_CH = 512
_NC = _L // _CH
_NEG = float("-inf")
_HIGH = jax.lax.Precision.HIGHEST


def _proj_kernel(q_ref, v_ref, vs_ref, wq_ref, wv_ref, wrow_ref,
                 vout_ref, qout_ref, idx_ref, ob_ref, m_scr, carry_scr):
    c = pl.program_id(1)
    prec_m = None

    Qc = jnp.dot(q_ref[0, 0], wq_ref[...], preferred_element_type=jnp.float32,
                 precision=prec_m)
    Vc = jnp.dot(v_ref[0, 0], wv_ref[...], preferred_element_type=jnp.float32,
                 precision=prec_m)
    vout_ref[0, 0] = Vc
    qout_ref[0, 0] = Qc

    Ks = jnp.dot(vs_ref[0, 0], wv_ref[...], preferred_element_type=jnp.float32,
                 precision=prec_m)

    @pl.when(c == 0)
    def _():
        carry_scr[...] = jnp.zeros_like(carry_scr)

    for h in range(_H):
        Ksh = Ks[:, h * _E:(h + 1) * _E]
        Qh = Qc[:, h * _E:(h + 1) * _E]
        St = jax.lax.dot_general(Ksh, Qh, (((1,), (1,)), ((), ())),
                                 preferred_element_type=jnp.float32,
                                 precision=prec_m)
        Mrow = jnp.max(St, axis=0) - jnp.sum(St, axis=0) * (1.0 / _L)
        m_scr[h, pl.ds(c * _CH, _CH)] = Mrow

    Vw = Vc * wrow_ref[0:1, :]
    R = (Vw[:, 0:128] + Vw[:, 128:256] + Vw[:, 256:384] + Vw[:, 384:512]
         + Vw[:, 512:640] + Vw[:, 640:768])
    Z = R[:, 0:_E] + R[:, _E:128]
    tri = (jax.lax.broadcasted_iota(jnp.int32, (_CH, _CH), 0)
           >= jax.lax.broadcasted_iota(jnp.int32, (_CH, _CH), 1)
           ).astype(jnp.float32)
    cumz = jnp.dot(tri, Z, preferred_element_type=jnp.float32,
                   precision=_HIGH) + carry_scr[0:1, :]
    carry_scr[0:1, :] = carry_scr[0:1, :] + jnp.sum(Z, axis=0, keepdims=True)
    ob_ref[0, 0] = cumz

    @pl.when(c == _NC - 1)
    def _():
        rows = jax.lax.broadcasted_iota(jnp.int32, (16, _L), 0)
        lanes = jax.lax.broadcasted_iota(jnp.int32, (16, _L), 1)
        cols = jax.lax.broadcasted_iota(jnp.int32, (16, 128), 1)
        M = jnp.where(rows < _H, m_scr[...], _NEG)

        def body(j, state):
            M, acc = state
            mval = jnp.max(M, axis=1, keepdims=True)
            cand = jnp.where(M == mval, lanes, _L)
            idx = jnp.min(cand, axis=1, keepdims=True)
            acc = jnp.where(cols == j, idx, acc)
            M = jnp.where(lanes == idx, _NEG, M)
            return M, acc

        _, acc = jax.lax.fori_loop(0, _U, body,
                                   (M, jnp.zeros((16, 128), jnp.int32)))
        idx_ref[0, 0] = acc


def _fix_kernel(v0_ref, v1_ref, q0_ref, i0_ref, i1_ref, ob_ref, w_ref,
                s_ref, out_ref):
    g = pl.program_id(1)
    V0 = v0_ref[0, 0]
    V1 = v1_ref[0, 0]
    Q0 = q0_ref[0, 0]
    tau = s_ref[0]
    delta = s_ref[1]
    scale = jnp.float32(1.0 / np.sqrt(_E))
    lanesL = jax.lax.broadcasted_iota(jnp.int32, (_U, _L), 1)

    acc = jnp.zeros((_L, _E), jnp.float32)
    for hh in range(2):
        h = 2 * g + hh
        wh = w_ref[h]
        sl = slice(hh * _E, (hh + 1) * _E)
        Q0h = Q0[:, sl]
        V0h = V0[:, sl]
        Ps, deltas = [], []
        for Vh, iref in ((V0h, i0_ref), (V1, i1_ref)):
            if Vh is V1:
                Vh = V1[:, sl]
            idx_col = iref[0, 0, pl.ds(h, 1), :][0, :_U].reshape(_U, 1)
            P = (lanesL == idx_col).astype(jnp.float32)
            SM = (lanesL <= idx_col).astype(jnp.float32)
            Qr = jnp.dot(P, Q0h, preferred_element_type=jnp.float32,
                         precision=_HIGH)
            S = jax.lax.dot_general(Qr, Vh, (((1,), (1,)), ((), ())),
                                    preferred_element_type=jnp.float32)
            S = (S * tau + delta) * scale
            S = S - jnp.max(S, axis=1, keepdims=True)
            Sexp = jnp.exp(S)
            A = Sexp / jnp.sum(Sexp, axis=1, keepdims=True)
            attn = jnp.dot(A, Vh, preferred_element_type=jnp.float32)
            cumsel = jnp.dot(SM, V0h, preferred_element_type=jnp.float32,
                             precision=_HIGH)
            Ps.append(P)
            deltas.append(attn - cumsel)
        m1 = jnp.sum(Ps[1], axis=0)
        t0 = jax.lax.dot_general(Ps[0], deltas[0], (((0,), (0,)), ((), ())),
                                 preferred_element_type=jnp.float32)
        t1 = jax.lax.dot_general(Ps[1], deltas[1], (((0,), (0,)), ((), ())),
                                 preferred_element_type=jnp.float32)
        acc = acc + wh * (t0 * (1.0 - m1)[:, None] + t1)

    @pl.when(g == 0)
    def _():
        out_ref[0] = ob_ref[0, 0]
    out_ref[0] += acc


def kernel(queries, keys, values, Wq, Wk, Wv, w_out, tau, delta):
    del keys, Wk
    wq2 = Wq.reshape(_D, _H * _E)
    wv2 = Wv.reshape(_D, _H * _E)
    vs = values[:, :, _IDX_SAMPLE, :]
    wrow = jnp.broadcast_to(jnp.repeat(w_out, _E)[None, :], (8, _H * _E))
    w_pad = jnp.concatenate([w_out, jnp.zeros((4,), jnp.float32)])
    scl = jnp.concatenate([tau, delta]).astype(jnp.float32)

    def bmap(p):
        return 1 - p // _T

    def tmap(p):
        return p % _T

    V, Q, topidx, out_base = pl.pallas_call(
        _proj_kernel,
        grid=(_B * _T, _NC),
        in_specs=[
            pl.BlockSpec((1, 1, _CH, _D), lambda p, c: (bmap(p), tmap(p), c, 0)),
            pl.BlockSpec((1, 1, _CH, _D), lambda p, c: (bmap(p), tmap(p), c, 0)),
            pl.BlockSpec((1, 1, _U, _D), lambda p, c: (bmap(p), tmap(p), 0, 0)),
            pl.BlockSpec((_D, _H * _E), lambda p, c: (0, 0)),
            pl.BlockSpec((_D, _H * _E), lambda p, c: (0, 0)),
            pl.BlockSpec((8, _H * _E), lambda p, c: (0, 0)),
        ],
        out_specs=[
            pl.BlockSpec((1, 1, _CH, _H * _E),
                         lambda p, c: (bmap(p), tmap(p), c, 0)),
            pl.BlockSpec((1, 1, _CH, _H * _E),
                         lambda p, c: (bmap(p), tmap(p), c, 0)),
            pl.BlockSpec((1, 1, 16, 128), lambda p, c: (bmap(p), tmap(p), 0, 0)),
            pl.BlockSpec((1, 1, _CH, _E), lambda p, c: (bmap(p), tmap(p), c, 0)),
        ],
        out_shape=[
            jax.ShapeDtypeStruct((_B, _T, _L, _H * _E), jnp.float32),
            jax.ShapeDtypeStruct((_B, _T, _L, _H * _E), jnp.float32),
            jax.ShapeDtypeStruct((_B, _T, 16, 128), jnp.int32),
            jax.ShapeDtypeStruct((_B, _T, _L, _E), jnp.float32),
        ],
        scratch_shapes=[
            pltpu.VMEM((16, _L), jnp.float32),
            pltpu.VMEM((8, _E), jnp.float32),
        ],
    )(queries, values, vs, wq2, wv2, wrow)

    out0 = pl.pallas_call(
        _fix_kernel,
        grid=(_T, _H // 2),
        in_specs=[
            pl.BlockSpec((1, 1, _L, 128), lambda t, g: (0, t, 0, g)),
            pl.BlockSpec((1, 1, _L, 128), lambda t, g: (1, t, 0, g)),
            pl.BlockSpec((1, 1, _L, 128), lambda t, g: (0, t, 0, g)),
            pl.BlockSpec((1, 1, 16, 128), lambda t, g: (0, t, 0, 0)),
            pl.BlockSpec((1, 1, 16, 128), lambda t, g: (1, t, 0, 0)),
            pl.BlockSpec((1, 1, _L, _E), lambda t, g: (0, t, 0, 0)),
            pl.BlockSpec(memory_space=pltpu.SMEM),
            pl.BlockSpec(memory_space=pltpu.SMEM),
        ],
        out_specs=pl.BlockSpec((1, _L, _E), lambda t, g: (t, 0, 0)),
        out_shape=jax.ShapeDtypeStruct((_T, _L, _E), jnp.float32),
    )(V, V, Q, topidx, topidx, out_base, w_pad, scl)

    return jnp.concatenate([out0[None], out_base[1:]], axis=0)
